# baseline (speedup 1.0000x reference)
import functools

import jax
import jax.numpy as jnp
from jax import lax
from jax.experimental import pallas as pl
from jax.experimental.pallas import tpu as pltpu
from jax.experimental.pallas import tpu_sc as plsc

B = 4096
H = 200
CA = 104
CB = H - CA
D = 32
C = 64
NBUF = 8

_info = plsc.get_sparse_core_info()
NC, NS = _info.num_cores, _info.num_subcores
NW = NC * NS
BPW = B // NW

_mesh = plsc.VectorSubcoreMesh(core_axis_name="c", subcore_axis_name="s")


@functools.partial(
    pl.kernel,
    mesh=_mesh,
    compiler_params=pltpu.CompilerParams(use_tc_tiling_on_sc=False),
    out_type=jax.ShapeDtypeStruct((B, D), jnp.float32),
    scratch_types=(
        [
            pltpu.VMEM((BPW, CA), jnp.int32),
            pltpu.VMEM((BPW, CB), jnp.int32),
            pltpu.VMEM((BPW, D), jnp.float32),
        ]
        + [pltpu.VMEM((CA, D), jnp.float32), pltpu.VMEM((CB, D), jnp.float32)]
        * (NBUF // 2)
        + [pltpu.SemaphoreType.DMA for _ in range(NBUF)]
    ),
)
def _sc_gather_sum(xa_hbm, xb_hbm, table_hbm, sums_hbm,
                   idxa_v, idxb_v, acc_v, b0, b1, b2, b3, b4, b5, b6, b7,
                   s0, s1, s2, s3, s4, s5, s6, s7):
    bufs = (b0, b1, b2, b3, b4, b5, b6, b7)
    sems = (s0, s1, s2, s3, s4, s5, s6, s7)
    idxs = (idxa_v, idxb_v)
    nrows = (CA, CB)
    wid = lax.axis_index("s") * NC + lax.axis_index("c")

    pltpu.sync_copy(xa_hbm.at[pl.ds(wid * BPW, BPW)], idxa_v)
    pltpu.sync_copy(xb_hbm.at[pl.ds(wid * BPW, BPW)], idxb_v)

    for k in range(NBUF):
        pltpu.async_copy(
            table_hbm.at[idxs[k % 2].at[k // 2]], bufs[k], sems[k]
        )

    def accum_chunk(slot, b, acc0, acc1):
        half = slot % 2
        pltpu.make_async_copy(
            table_hbm.at[idxs[half].at[b]], bufs[slot], sems[slot]
        ).wait()
        for r in range(nrows[half]):
            acc0 = acc0 + bufs[slot][r, pl.ds(0, 16)]
            acc1 = acc1 + bufs[slot][r, pl.ds(16, 16)]

        @pl.when(b + NBUF // 2 < BPW)
        def _():
            pltpu.async_copy(
                table_hbm.at[idxs[half].at[b + NBUF // 2]], bufs[slot], sems[slot]
            )

        return acc0, acc1

    def step(o, carry):
        for p in range(4):
            b = 4 * o + p
            zero = jnp.zeros((16,), jnp.float32)
            acc0, acc1 = zero, zero
            for half in range(2):
                slot = 2 * p + half
                acc0, acc1 = accum_chunk(slot, b, acc0, acc1)
            acc_v[b, pl.ds(0, 16)] = acc0
            acc_v[b, pl.ds(16, 16)] = acc1
        return carry

    lax.fori_loop(0, BPW // 4, step, 0)

    pltpu.sync_copy(acc_v, sums_hbm.at[pl.ds(wid * BPW, BPW)])


_TCB = 512


def _tc_head(x_ref, sums_ref, w_ref, b_ref, out_ref):
    cnt = jnp.sum((x_ref[...] != 0).astype(jnp.float32), axis=1, keepdims=True)
    avg = sums_ref[...] / jnp.maximum(cnt, 1.0)
    out_ref[...] = (
        jnp.dot(avg, w_ref[...], preferred_element_type=jnp.float32) + b_ref[...]
    )


def kernel(x, emb_table, fc_w, fc_b):
    x = x.astype(jnp.int32)
    sums = _sc_gather_sum(x[:, :CA], x[:, CA:], emb_table)
    return pl.pallas_call(
        _tc_head,
        grid=(B // _TCB,),
        in_specs=[
            pl.BlockSpec((_TCB, H), lambda i: (i, 0)),
            pl.BlockSpec((_TCB, D), lambda i: (i, 0)),
            pl.BlockSpec((D, C), lambda i: (0, 0)),
            pl.BlockSpec((1, C), lambda i: (0, 0)),
        ],
        out_specs=pl.BlockSpec((_TCB, C), lambda i: (i, 0)),
        out_shape=jax.ShapeDtypeStruct((B, C), jnp.float32),
    )(x, sums, fc_w, fc_b.reshape(1, C))


# speedup vs baseline: 1.0190x; 1.0190x over previous
import functools

import jax
import jax.numpy as jnp
from jax import lax
from jax.experimental import pallas as pl
from jax.experimental.pallas import tpu as pltpu
from jax.experimental.pallas import tpu_sc as plsc

B = 4096
H = 200
CA = 104
CB = H - CA
D = 32
C = 64
NBUF = 4

_info = plsc.get_sparse_core_info()
NC, NS = _info.num_cores, _info.num_subcores
NW = NC * NS
BPW = B // NW

_mesh = plsc.VectorSubcoreMesh(core_axis_name="c", subcore_axis_name="s")


@functools.partial(
    pl.kernel,
    mesh=_mesh,
    compiler_params=pltpu.CompilerParams(use_tc_tiling_on_sc=False),
    out_type=jax.ShapeDtypeStruct((B, D), jnp.float32),
    scratch_types=(
        [
            pltpu.VMEM((BPW, CA), jnp.int32),
            pltpu.VMEM((BPW, CB), jnp.int32),
            pltpu.VMEM((BPW, D), jnp.float32),
        ]
        + [pltpu.VMEM((CA, D), jnp.float32), pltpu.VMEM((CB, D), jnp.float32)]
        * (NBUF // 2)
        + [pltpu.SemaphoreType.DMA for _ in range(NBUF)]
    ),
)
def _sc_gather_sum(xa_hbm, xb_hbm, table_hbm, sums_hbm,
                   idxa_v, idxb_v, acc_v, b0, b1, b2, b3, s0, s1, s2, s3):
    bufs = (b0, b1, b2, b3)
    sems = (s0, s1, s2, s3)
    idxs = (idxa_v, idxb_v)
    nrows = (CA, CB)
    wid = lax.axis_index("s") * NC + lax.axis_index("c")

    pltpu.sync_copy(xa_hbm.at[pl.ds(wid * BPW, BPW)], idxa_v)
    pltpu.sync_copy(xb_hbm.at[pl.ds(wid * BPW, BPW)], idxb_v)

    for k in range(NBUF):
        pltpu.async_copy(
            table_hbm.at[idxs[k % 2].at[k // 2]], bufs[k], sems[k]
        )

    def accum_chunk(slot, b, acc0, acc1):
        half = slot % 2
        pltpu.make_async_copy(
            table_hbm.at[idxs[half].at[b]], bufs[slot], sems[slot]
        ).wait()
        for r in range(nrows[half]):
            acc0 = acc0 + bufs[slot][r, pl.ds(0, 16)]
            acc1 = acc1 + bufs[slot][r, pl.ds(16, 16)]

        @pl.when(b + 2 < BPW)
        def _():
            pltpu.async_copy(
                table_hbm.at[idxs[half].at[b + 2]], bufs[slot], sems[slot]
            )

        return acc0, acc1

    def step(o, carry):
        for p in range(2):
            b = 2 * o + p
            zero = jnp.zeros((16,), jnp.float32)
            acc0, acc1 = zero, zero
            for half in range(2):
                slot = 2 * p + half
                acc0, acc1 = accum_chunk(slot, b, acc0, acc1)
            acc_v[b, pl.ds(0, 16)] = acc0
            acc_v[b, pl.ds(16, 16)] = acc1
        return carry

    lax.fori_loop(0, BPW // 2, step, 0)

    pltpu.sync_copy(acc_v, sums_hbm.at[pl.ds(wid * BPW, BPW)])


_TCB = 512


def _tc_head(x_ref, sums_ref, w_ref, b_ref, out_ref):
    cnt = jnp.sum((x_ref[...] != 0).astype(jnp.float32), axis=1, keepdims=True)
    avg = sums_ref[...] / jnp.maximum(cnt, 1.0)
    out_ref[...] = (
        jnp.dot(avg, w_ref[...], preferred_element_type=jnp.float32) + b_ref[...]
    )


def kernel(x, emb_table, fc_w, fc_b):
    x = x.astype(jnp.int32)
    sums = _sc_gather_sum(x[:, :CA], x[:, CA:], emb_table)
    return pl.pallas_call(
        _tc_head,
        grid=(B // _TCB,),
        in_specs=[
            pl.BlockSpec((_TCB, H), lambda i: (i, 0)),
            pl.BlockSpec((_TCB, D), lambda i: (i, 0)),
            pl.BlockSpec((D, C), lambda i: (0, 0)),
            pl.BlockSpec((1, C), lambda i: (0, 0)),
        ],
        out_specs=pl.BlockSpec((_TCB, C), lambda i: (i, 0)),
        out_shape=jax.ShapeDtypeStruct((B, C), jnp.float32),
    )(x, sums, fc_w, fc_b.reshape(1, C))
